# Initial kernel scaffold; baseline (speedup 1.0000x reference)
import functools

import jax
import jax.numpy as jnp
from jax import lax
from jax.experimental import pallas as pl
from jax.experimental.pallas import tpu as pltpu
from jax.experimental.pallas import tpu_sc as plsc

D_MODEL = 32
NUM_CORES = 2
NUM_SUBCORES = 16
NUM_WORKERS = NUM_CORES * NUM_SUBCORES

CHUNK = 1024


def _gather_body(x_hbm, table_hbm, out_hbm, idx_v, rows_v, sem, *, rows_per_worker):
  wid = lax.axis_index("s") * NUM_CORES + lax.axis_index("c")
  base_w = wid * rows_per_worker
  n_chunks = rows_per_worker // CHUNK

  @pl.loop(0, n_chunks)
  def _chunk(i):
    base = base_w + i * CHUNK
    pltpu.sync_copy(x_hbm.at[pl.ds(base, CHUNK)], idx_v)
    pltpu.async_copy(table_hbm.at[idx_v], rows_v, sem).wait()
    pltpu.sync_copy(rows_v, out_hbm.at[pl.ds(base, CHUNK)])


def kernel(x, table):
  b, l = x.shape
  n = b * l
  rows_per_worker = n // NUM_WORKERS
  x_flat = x.reshape(n).astype(jnp.int32)

  mesh = plsc.VectorSubcoreMesh(core_axis_name="c", subcore_axis_name="s")
  out = pl.kernel(
      functools.partial(_gather_body, rows_per_worker=rows_per_worker),
      out_type=jax.ShapeDtypeStruct((n, D_MODEL), jnp.float32),
      mesh=mesh,
      scratch_types=[
          pltpu.VMEM((CHUNK,), jnp.int32),
          pltpu.VMEM((CHUNK, D_MODEL), jnp.float32),
          pltpu.SemaphoreType.DMA,
      ],
  )(x_flat, table)
  return out.reshape(b, l, D_MODEL)


# speedup vs baseline: 1.0952x; 1.0952x over previous
import functools

import jax
import jax.numpy as jnp
from jax import lax
from jax.experimental import pallas as pl
from jax.experimental.pallas import tpu as pltpu
from jax.experimental.pallas import tpu_sc as plsc

D_MODEL = 32
NUM_CORES = 2
NUM_SUBCORES = 16
NUM_WORKERS = NUM_CORES * NUM_SUBCORES

CHUNK = 1024


def _gather_body(x_hbm, table_hbm, out_hbm, idx_v, rows_v, sem, *, rows_per_worker):
  wid = lax.axis_index("s") * NUM_CORES + lax.axis_index("c")
  base_w = wid * rows_per_worker
  n_chunks = rows_per_worker // CHUNK

  @pl.loop(0, n_chunks)
  def _chunk(i):
    base = base_w + i * CHUNK
    pltpu.sync_copy(x_hbm.at[pl.ds(base, CHUNK)], idx_v)
    pltpu.async_copy(table_hbm.at[idx_v], rows_v, sem).wait()
    pltpu.sync_copy(rows_v, out_hbm.at[pl.ds(base, CHUNK)])


def kernel(x, table):
  b, l = x.shape
  n = b * l
  rows_per_worker = n // NUM_WORKERS
  x_flat = x.reshape(n).astype(jnp.int32)

  mesh = plsc.VectorSubcoreMesh(core_axis_name="c", subcore_axis_name="s")
  out = pl.kernel(
      functools.partial(_gather_body, rows_per_worker=rows_per_worker),
      out_type=jax.ShapeDtypeStruct((n, D_MODEL), jnp.float32),
      mesh=mesh,
      scratch_types=[
          pltpu.VMEM((CHUNK,), jnp.int32),
          pltpu.VMEM((CHUNK, D_MODEL), jnp.float32),
          pltpu.SemaphoreType.DMA,
      ],
      compiler_params=pltpu.CompilerParams(use_tc_tiling_on_sc=False),
  )(x_flat, table)
  return out.reshape(b, l, D_MODEL)


# speedup vs baseline: 1.1077x; 1.0114x over previous
import functools

import jax
import jax.numpy as jnp
from jax import lax
from jax.experimental import pallas as pl
from jax.experimental.pallas import tpu as pltpu
from jax.experimental.pallas import tpu_sc as plsc

D_MODEL = 32
NUM_CORES = 2
NUM_SUBCORES = 16
NUM_WORKERS = NUM_CORES * NUM_SUBCORES

CHUNK = 1280
NBUF = 2


def _gather_body(x_hbm, table_hbm, out_hbm, idx_v, rows_v, idx_sems,
                 gat_sems, out_sems, *, rows_per_worker):
  wid = lax.axis_index("s") * NUM_CORES + lax.axis_index("c")
  base_w = wid * rows_per_worker
  n_outer = rows_per_worker // (CHUNK * NBUF)

  @pl.loop(0, n_outer)
  def _outer(o):
    base_o = base_w + o * (CHUNK * NBUF)
    for b in range(NBUF):
      base = base_o + b * CHUNK

      @pl.when(o > 0)
      def _():
        pltpu.make_async_copy(
            rows_v.at[b], out_hbm.at[pl.ds(base, CHUNK)], out_sems.at[b]
        ).wait()

      pltpu.async_copy(x_hbm.at[pl.ds(base, CHUNK)], idx_v.at[b],
                       idx_sems.at[b])
    for b in range(NBUF):
      base = base_o + b * CHUNK
      pltpu.make_async_copy(x_hbm.at[pl.ds(base, CHUNK)], idx_v.at[b],
                            idx_sems.at[b]).wait()
      pltpu.async_copy(table_hbm.at[idx_v.at[b]], rows_v.at[b],
                       gat_sems.at[b])
    for b in range(NBUF):
      base = base_o + b * CHUNK
      pltpu.make_async_copy(table_hbm.at[idx_v.at[b]], rows_v.at[b],
                            gat_sems.at[b]).wait()
      pltpu.async_copy(rows_v.at[b], out_hbm.at[pl.ds(base, CHUNK)],
                       out_sems.at[b])

  last_base = base_w + (n_outer - 1) * (CHUNK * NBUF)
  for b in range(NBUF):
    pltpu.make_async_copy(
        rows_v.at[b],
        out_hbm.at[pl.ds(last_base + b * CHUNK, CHUNK)],
        out_sems.at[b],
    ).wait()


def kernel(x, table):
  b, l = x.shape
  n = b * l
  rows_per_worker = n // NUM_WORKERS
  x_flat = x.reshape(n).astype(jnp.int32)

  mesh = plsc.VectorSubcoreMesh(core_axis_name="c", subcore_axis_name="s")
  out = pl.kernel(
      functools.partial(_gather_body, rows_per_worker=rows_per_worker),
      out_type=jax.ShapeDtypeStruct((n, D_MODEL), jnp.float32),
      mesh=mesh,
      scratch_types=[
          pltpu.VMEM((NBUF, CHUNK), jnp.int32),
          pltpu.VMEM((NBUF, CHUNK, D_MODEL), jnp.float32),
          pltpu.SemaphoreType.DMA((NBUF,)),
          pltpu.SemaphoreType.DMA((NBUF,)),
          pltpu.SemaphoreType.DMA((NBUF,)),
      ],
      compiler_params=pltpu.CompilerParams(use_tc_tiling_on_sc=False),
  )(x_flat, table)
  return out.reshape(b, l, D_MODEL)


# speedup vs baseline: 5.0543x; 4.5628x over previous
import functools

import jax
import jax.numpy as jnp
from jax import lax
from jax.experimental import pallas as pl
from jax.experimental.pallas import tpu as pltpu
from jax.experimental.pallas import tpu_sc as plsc

D_MODEL = 32
MAX_LEN = 1000000
MAIN_LEN = (MAX_LEN // 128) * 128
TAIL_LEN = 128
PLANE_PAD = MAIN_LEN + TAIL_LEN
NUM_CORES = 2
NUM_SUBCORES = 16
PLANES_PER_CORE = D_MODEL // NUM_CORES
NHALF = 2
NSEG_HALF = 5


def _stage_plane(tablet_hbm, tail_hbm, plane_sp, psem, d):
  pltpu.async_copy(tablet_hbm.at[d, pl.ds(0, MAIN_LEN)],
                   plane_sp.at[pl.ds(0, MAIN_LEN)], psem)
  pltpu.async_copy(tail_hbm.at[d], plane_sp.at[pl.ds(MAIN_LEN, TAIL_LEN)],
                   psem)


def _wait_plane(tablet_hbm, tail_hbm, plane_sp, psem, d):
  pltpu.make_async_copy(tablet_hbm.at[d, pl.ds(0, MAIN_LEN)],
                        plane_sp.at[pl.ds(0, MAIN_LEN)], psem).wait()
  pltpu.make_async_copy(tail_hbm.at[d],
                        plane_sp.at[pl.ds(MAIN_LEN, TAIL_LEN)], psem).wait()


def _plane_body(xt_hbm, tablet_hbm, tail_hbm, out_hbm,
                idx0, idx1, dest0, dest1, plane_sp,
                psem, gsem0, gsem1, wsem0, wsem1, isem,
                *, b, l):
  c = lax.axis_index("c")
  s = lax.axis_index("s")
  b_per_tile = b // NUM_SUBCORES
  b0 = s * b_per_tile
  l_half = l // NHALF
  seg_rows = l_half // NSEG_HALF
  seg = seg_rows * b_per_tile
  nseg = l // seg_rows
  idx_refs = (idx0, idx1)
  dest_refs = (dest0, dest1)
  gsems = (gsem0, gsem1)
  wsems = (wsem0, wsem1)
  d_base = c * PLANES_PER_CORE

  def seg_idx_slice(g):
    return idx_refs[g // NSEG_HALF].at[
        pl.ds((g % NSEG_HALF) * seg, seg)]

  def wb_copies(g, d, dbuf, wsem):
    for r in range(seg_rows):
      li = g * seg_rows + r
      yield pltpu.make_async_copy(
          dbuf.at[pl.ds(r * b_per_tile, b_per_tile)],
          out_hbm.at[li, d, pl.ds(b0, b_per_tile)], wsem)

  for li in range(l):
    pltpu.async_copy(
        xt_hbm.at[li, pl.ds(b0, b_per_tile)],
        idx_refs[li // l_half].at[pl.ds((li % l_half) * b_per_tile,
                                        b_per_tile)], isem)
  for li in range(l):
    pltpu.make_async_copy(
        xt_hbm.at[li, pl.ds(b0, b_per_tile)],
        idx_refs[li // l_half].at[pl.ds((li % l_half) * b_per_tile,
                                        b_per_tile)], isem).wait()

  @pl.when(s == 0)
  def _():
    _stage_plane(tablet_hbm, tail_hbm, plane_sp, psem, d_base)

  @pl.loop(0, PLANES_PER_CORE)
  def _plane(j):
    d = d_base + j

    @pl.when(s == 0)
    def _():
      _wait_plane(tablet_hbm, tail_hbm, plane_sp, psem, d)

    plsc.subcore_barrier()

    for g in range(nseg):
      dbuf = dest_refs[g % 2]
      wsem = wsems[g % 2]
      gsem = gsems[g % 2]

      def drain(g=g, dbuf=dbuf, wsem=wsem):
        for cp in wb_copies(g, d, dbuf, wsem):
          cp.wait()

      if g < 2:
        @pl.when(j > 0)
        def _(drain=drain):
          drain()
      else:
        drain()

      pltpu.async_copy(plane_sp.at[seg_idx_slice(g)], dbuf, gsem)

      if g > 0:
        pg = g - 1
        pltpu.make_async_copy(plane_sp.at[seg_idx_slice(pg)],
                              dest_refs[pg % 2], gsems[pg % 2]).wait()
        for cp in wb_copies(pg, d, dest_refs[pg % 2], wsems[pg % 2]):
          cp.start()

    pg = nseg - 1
    pltpu.make_async_copy(plane_sp.at[seg_idx_slice(pg)],
                          dest_refs[pg % 2], gsems[pg % 2]).wait()
    for cp in wb_copies(pg, d, dest_refs[pg % 2], wsems[pg % 2]):
      cp.start()

    plsc.subcore_barrier()

    @pl.when(jnp.logical_and(s == 0, j + 1 < PLANES_PER_CORE))
    def _():
      _stage_plane(tablet_hbm, tail_hbm, plane_sp, psem, d + 1)

  d_last = d_base + PLANES_PER_CORE - 1
  for g in (nseg - 2, nseg - 1):
    for cp in wb_copies(g, d_last, dest_refs[g % 2], wsems[g % 2]):
      cp.wait()


def kernel(x, table):
  b, l = x.shape
  xt = x.T.astype(jnp.int32)
  tablet = table.T
  tail = jnp.concatenate(
      [table[MAIN_LEN:], jnp.zeros((TAIL_LEN - (MAX_LEN - MAIN_LEN), D_MODEL),
                                   jnp.float32)], axis=0)
  tail_t = tail.T
  b_per_tile = b // NUM_SUBCORES
  l_half = l // NHALF
  seg_elems = (l_half // NSEG_HALF) * b_per_tile

  mesh = plsc.VectorSubcoreMesh(core_axis_name="c", subcore_axis_name="s")
  out_t = pl.kernel(
      functools.partial(_plane_body, b=b, l=l),
      out_type=jax.ShapeDtypeStruct((l, D_MODEL, b), jnp.float32),
      mesh=mesh,
      scratch_types=[
          pltpu.VMEM((l_half * b_per_tile,), jnp.int32),
          pltpu.VMEM((l_half * b_per_tile,), jnp.int32),
          pltpu.VMEM((seg_elems,), jnp.float32),
          pltpu.VMEM((seg_elems,), jnp.float32),
          pltpu.VMEM_SHARED((PLANE_PAD,), jnp.float32),
          pltpu.SemaphoreType.DMA,
          pltpu.SemaphoreType.DMA,
          pltpu.SemaphoreType.DMA,
          pltpu.SemaphoreType.DMA,
          pltpu.SemaphoreType.DMA,
          pltpu.SemaphoreType.DMA,
      ],
      compiler_params=pltpu.CompilerParams(use_tc_tiling_on_sc=True),
  )(xt, tablet, tail_t)
  return out_t.transpose(2, 0, 1)
